# baseline (speedup 1.0000x reference)
import functools

import jax
import jax.numpy as jnp
from jax import lax
from jax.experimental import pallas as pl
from jax.experimental.pallas import tpu as pltpu
from jax.experimental.pallas import tpu_sc as plsc

D = 1024
L = 16
EPS = 1e-05
SEQ = 4096
C = 16
NB = D // L
HR = 8

_GATHER_DNUMS = lax.GatherDimensionNumbers(
    offset_dims=(), collapsed_slice_dims=(0,), start_index_map=(0,))


def _lane_pick(v, idx):
    return lax.gather(v, idx[:, None], _GATHER_DNUMS, slice_sizes=(1,),
                      mode=lax.GatherScatterMode.PROMISE_IN_BOUNDS)


def _lane_sum(v, shuf_idx):
    for ix in shuf_idx:
        v = v + _lane_pick(v, ix)
    return v


def _rsqrt(v):
    i = plsc.bitcast(v, jnp.int32)
    y = plsc.bitcast(jnp.int32(0x5F3759DF) - (i >> 1), jnp.float32)
    for _ in range(3):
        y = y * (1.5 - 0.5 * v * y * y)
    return y


@functools.lru_cache(maxsize=None)
def _make_sc_kernel(n_rows):
    info = plsc.get_sparse_core_info()
    nw = info.num_cores * info.num_subcores
    per_w = n_rows // nw
    n_g = per_w // C
    mesh = plsc.VectorSubcoreMesh(core_axis_name="c", subcore_axis_name="s")

    @functools.partial(
        pl.kernel,
        mesh=mesh,
        out_type=jax.ShapeDtypeStruct((n_rows, D), jnp.float32),
        compiler_params=pltpu.CompilerParams(needs_layout_passes=False,
                                             use_tc_tiling_on_sc=False),
        scratch_types=[
            pltpu.VMEM((per_w,), jnp.int32),
            pltpu.VMEM((C, D), jnp.float32),
            pltpu.VMEM((C, D), jnp.float32),
            pltpu.VMEM((C, D), jnp.float32),
            pltpu.VMEM((C, D), jnp.float32),
            pltpu.VMEM((C,), jnp.int32),
            pltpu.VMEM((D,), jnp.float32),
            pltpu.VMEM((D,), jnp.float32),
            pltpu.SemaphoreType.DMA,
            pltpu.SemaphoreType.DMA,
            pltpu.SemaphoreType.DMA,
            pltpu.SemaphoreType.DMA,
            pltpu.SemaphoreType.DMA,
            pltpu.SemaphoreType.DMA,
        ],
    )
    def k(ids_hbm, word_hbm, pos_hbm, type_hbm, w_hbm, b_hbm, out_hbm,
          idx_v, comb_b0, comb_b1, out_b0, out_b1,
          tz_v, w_v, b_v, semw0, semw1, semp0, semp1, semo0, semo1):
        comb_bufs = (comb_b0, comb_b1)
        out_bufs = (out_b0, out_b1)
        semw = (semw0, semw1)
        semp = (semp0, semp1)
        semo = (semo0, semo1)

        wid = lax.axis_index("s") * info.num_cores + lax.axis_index("c")
        base = wid * per_w
        s0 = lax.rem(base, SEQ)
        pltpu.sync_copy(ids_hbm.at[pl.ds(base, per_w)], idx_v)
        pltpu.sync_copy(w_hbm, w_v)
        pltpu.sync_copy(b_hbm, b_v)
        tz_v[...] = jnp.zeros((C,), jnp.int32)

        shuf_idx = [lax.iota(jnp.int32, L) ^ sh for sh in (8, 4, 2, 1)]
        zf = jnp.zeros((L,), jnp.float32)

        def pos_copy(g, p):
            row0 = pl.multiple_of(g * C, C)
            return pltpu.make_async_copy(
                pos_hbm.at[pl.ds(s0 + row0, C)], comb_bufs[p], semp[p])

        def gadd_start(g, p):
            row0 = pl.multiple_of(g * C, C)
            pltpu.async_copy(
                word_hbm.at[idx_v.at[pl.ds(row0, C)]], comb_bufs[p],
                semw[p], add=True)
            pltpu.async_copy(
                type_hbm.at[tz_v], comb_bufs[p], semw[p], add=True)

        def gadd_wait(g, p):
            row0 = pl.multiple_of(g * C, C)
            pltpu.make_async_copy(
                word_hbm.at[idx_v.at[pl.ds(row0, C)]], comb_bufs[p],
                semw[p]).wait()
            pltpu.make_async_copy(
                type_hbm.at[tz_v], comb_bufs[p], semw[p]).wait()

        def out_copy(g, p):
            row0 = pl.multiple_of(base + g * C, C)
            return pltpu.make_async_copy(
                out_bufs[p], out_hbm.at[pl.ds(row0, C)], semo[p])

        pos_copy(0, 0).start()
        pos_copy(1, 1).start()
        pos_copy(0, 0).wait()
        gadd_start(0, 0)

        def pass1_half(cv, r8):
            def blk(bi, carry):
                c0 = pl.multiple_of(bi * L, L)
                vss, vqs = carry
                nss, nqs = [], []
                for r in range(HR):
                    x = cv[r8 + r, pl.ds(c0, L)]
                    nss.append(vss[r] + x)
                    nqs.append(vqs[r] + x * x)
                return tuple(nss), tuple(nqs)

            return plsc.parallel_loop(
                0, NB, carry=((zf,) * HR, (zf,) * HR))(blk)

        def pass2_half(cv, ov, ms, rs, r8):
            def blk(bi, carry):
                c0 = pl.multiple_of(bi * L, L)
                wch = w_v[pl.ds(c0, L)]
                bch = b_v[pl.ds(c0, L)]
                for r in range(HR):
                    x = cv[r8 + r, pl.ds(c0, L)]
                    ov[r8 + r, pl.ds(c0, L)] = (
                        (x - ms[r]) * rs[r] * wch + bch)
                return carry

            plsc.parallel_loop(0, NB, carry=jnp.int32(0))(blk)

        def process(g, p):
            @pl.when(g >= 2)
            def _():
                out_copy(g, p).wait()

            gadd_wait(g, p)

            @pl.when(g + 1 < n_g)
            def _():
                pos_copy(g + 1, 1 - p).wait()
                gadd_start(g + 1, 1 - p)

            cv, ov = comb_bufs[p], out_bufs[p]
            stats = []
            for r8 in (0, HR):
                vss, vqs = pass1_half(cv, r8)
                for r in range(HR):
                    ts = _lane_sum(vss[r], shuf_idx)
                    tq = _lane_sum(vqs[r], shuf_idx)
                    m = ts * (1.0 / D)
                    var = tq * (1.0 / D) - m * m
                    stats.append((m, _rsqrt(var + EPS)))

            for r8 in (0, HR):
                ms = [stats[r8 + r][0] for r in range(HR)]
                rs = [stats[r8 + r][1] for r in range(HR)]
                pass2_half(cv, ov, ms, rs, r8)

            @pl.when(g + 2 < n_g)
            def _():
                pos_copy(g + 2, p).start()

            out_copy(g, p).start()

        def outer(go, carry):
            process(2 * go, 0)
            process(2 * go + 1, 1)
            return carry

        lax.fori_loop(0, n_g // 2, outer, 0)
        out_copy(n_g - 2, 0).wait()
        out_copy(n_g - 1, 1).wait()

    return k


def kernel(input_ids, word_table, pos_table, type_table, ln_weight, ln_bias):
    b, s = input_ids.shape
    ids_flat = jnp.reshape(input_ids.astype(jnp.int32), (b * s,))
    type_2d = jnp.reshape(type_table, (1, D))
    k = _make_sc_kernel(b * s)
    out = k(ids_flat, word_table, pos_table, type_2d, ln_weight, ln_bias)
    return jnp.reshape(out, (b, s, D))


# speedup vs baseline: 1.5933x; 1.5933x over previous
import functools

import jax
import jax.numpy as jnp
from jax import lax
from jax.experimental import pallas as pl
from jax.experimental.pallas import tpu as pltpu
from jax.experimental.pallas import tpu_sc as plsc

D = 1024
L = 16
EPS = 1e-05
SEQ = 4096
C = 16
NB = D // L
QR = 4

_GATHER_DNUMS = lax.GatherDimensionNumbers(
    offset_dims=(), collapsed_slice_dims=(0,), start_index_map=(0,))


def _lane_pick(v, idx):
    return lax.gather(v, idx[:, None], _GATHER_DNUMS, slice_sizes=(1,),
                      mode=lax.GatherScatterMode.PROMISE_IN_BOUNDS)


def _rsqrt(v):
    i = plsc.bitcast(v, jnp.int32)
    y = plsc.bitcast(jnp.int32(0x5F3759DF) - (i >> 1), jnp.float32)
    for _ in range(3):
        y = y * (1.5 - 0.5 * v * y * y)
    return y


@functools.lru_cache(maxsize=None)
def _make_sc_kernel(n_rows):
    info = plsc.get_sparse_core_info()
    nw = info.num_cores * info.num_subcores
    per_w = n_rows // nw
    n_g = per_w // C
    assert n_g == 32
    mesh = plsc.VectorSubcoreMesh(core_axis_name="c", subcore_axis_name="s")

    @functools.partial(
        pl.kernel,
        mesh=mesh,
        out_type=jax.ShapeDtypeStruct((n_rows, D), jnp.float32),
        compiler_params=pltpu.CompilerParams(needs_layout_passes=False,
                                             use_tc_tiling_on_sc=False),
        scratch_types=[
            pltpu.VMEM((per_w,), jnp.int32),
            pltpu.VMEM((C, D), jnp.float32),
            pltpu.VMEM((C, D), jnp.float32),
            pltpu.VMEM((C, D), jnp.float32),
            pltpu.VMEM((C, D), jnp.float32),
            pltpu.VMEM((C, D), jnp.float32),
            pltpu.VMEM((C, D), jnp.float32),
            pltpu.VMEM((C, D), jnp.float32),
            pltpu.VMEM((2 * C, L), jnp.float32),
            pltpu.VMEM((D,), jnp.float32),
            pltpu.VMEM((D,), jnp.float32),
            pltpu.VMEM((D,), jnp.float32),
            pltpu.SemaphoreType.DMA,
            pltpu.SemaphoreType.DMA,
            pltpu.SemaphoreType.DMA,
            pltpu.SemaphoreType.DMA,
            pltpu.SemaphoreType.DMA,
            pltpu.SemaphoreType.DMA,
            pltpu.SemaphoreType.DMA,
        ],
    )
    def k(ids_hbm, word_hbm, pos_hbm, type_hbm, w_hbm, b_hbm, out_hbm,
          idx_v, word_b0, word_b1, pos_b0, pos_b1, ob0, ob1, ob2, st_v,
          type_v, w_v, b_v, semw0, semw1, semp0, semp1, semo0, semo1,
          semo2):
        word_bufs = (word_b0, word_b1)
        pos_bufs = (pos_b0, pos_b1)
        out_bufs = (ob0, ob1, ob2)
        semw = (semw0, semw1)
        semp = (semp0, semp1)
        semo = (semo0, semo1, semo2)

        wid = lax.axis_index("s") * info.num_cores + lax.axis_index("c")
        base = wid * per_w
        s0 = lax.rem(base, SEQ)
        pltpu.sync_copy(ids_hbm.at[pl.ds(base, per_w)], idx_v)
        pltpu.sync_copy(type_hbm, type_v)
        pltpu.sync_copy(w_hbm, w_v)
        pltpu.sync_copy(b_hbm, b_v)

        cjj = [jnp.full((L,), jj, jnp.int32) for jj in range(L)]
        lanes = lax.iota(jnp.int32, L)
        zf = jnp.zeros((L,), jnp.float32)

        def word_gather(g, wp):
            row0 = pl.multiple_of(g * C, C)
            return pltpu.make_async_copy(
                word_hbm.at[idx_v.at[pl.ds(row0, C)]], word_bufs[wp],
                semw[wp])

        def pos_copy(g, wp):
            row0 = pl.multiple_of(g * C, C)
            return pltpu.make_async_copy(
                pos_hbm.at[pl.ds(s0 + row0, C)], pos_bufs[wp], semp[wp])

        def out_copy(g, op):
            row0 = pl.multiple_of(base + g * C, C)
            return pltpu.make_async_copy(
                out_bufs[op], out_hbm.at[pl.ds(row0, C)], semo[op])

        def start_gather(g, wp):
            word_gather(g, wp).start()
            pos_copy(g, wp).start()

        def p1_quarter(wp, op, q0):
            wv, pv, ov = word_bufs[wp], pos_bufs[wp], out_bufs[op]

            def blk(bi, carry):
                c0 = pl.multiple_of(bi * L, L)
                tch = type_v[pl.ds(c0, L)]
                vss, vqs = carry
                nss, nqs = [], []
                for r in range(QR):
                    x = wv[q0 + r, pl.ds(c0, L)] + pv[q0 + r, pl.ds(c0, L)] \
                        + tch
                    ov[q0 + r, pl.ds(c0, L)] = x
                    nss.append(vss[r] + x)
                    nqs.append(vqs[r] + x * x)
                return tuple(nss), tuple(nqs)

            return plsc.parallel_loop(
                0, NB, carry=((zf,) * QR, (zf,) * QR))(blk)

        def p2_quarter(op, ms, rs, q0):
            ov = out_bufs[op]

            def blk(bi, carry):
                c0 = pl.multiple_of(bi * L, L)
                wch = w_v[pl.ds(c0, L)]
                bch = b_v[pl.ds(c0, L)]
                for r in range(QR):
                    x = ov[q0 + r, pl.ds(c0, L)]
                    ov[q0 + r, pl.ds(c0, L)] = (
                        (x - ms[r]) * rs[r] * wch + bch)
                return carry

            plsc.parallel_loop(0, NB, carry=jnp.int32(0))(blk)

        def merged_quarter(wp, op_cur, op_prev, ms, rs, q0):
            wv, pv = word_bufs[wp], pos_bufs[wp]
            ov_c, ov_p = out_bufs[op_cur], out_bufs[op_prev]

            def blk(bi, carry):
                c0 = pl.multiple_of(bi * L, L)
                tch = type_v[pl.ds(c0, L)]
                wch = w_v[pl.ds(c0, L)]
                bch = b_v[pl.ds(c0, L)]
                vss, vqs = carry
                nss, nqs = [], []
                for r in range(QR):
                    x = wv[q0 + r, pl.ds(c0, L)] + pv[q0 + r, pl.ds(c0, L)] \
                        + tch
                    ov_c[q0 + r, pl.ds(c0, L)] = x
                    nss.append(vss[r] + x)
                    nqs.append(vqs[r] + x * x)
                    xo = ov_p[q0 + r, pl.ds(c0, L)]
                    ov_p[q0 + r, pl.ds(c0, L)] = (
                        (xo - ms[r]) * rs[r] * wch + bch)
                return tuple(nss), tuple(nqs)

            return plsc.parallel_loop(
                0, NB, carry=((zf,) * QR, (zf,) * QR))(blk)

        def batched_stats(acc_list):
            for r, (vs, vq) in enumerate(acc_list):
                st_v[r, :] = vs
                st_v[C + r, :] = vq
            tsum = zf
            qsum = zf
            for kcol in range(L):
                tsum = tsum + plsc.load_gather(st_v, [lanes, cjj[kcol]])
                qsum = qsum + plsc.load_gather(
                    st_v, [lanes + C, cjj[kcol]])
            mean_v = tsum * (1.0 / D)
            var_v = qsum * (1.0 / D) - mean_v * mean_v
            return mean_v, _rsqrt(var_v + EPS)

        def splats(mean_v, rstd_v):
            ms = [_lane_pick(mean_v, cjj[r]) for r in range(C)]
            rs = [_lane_pick(rstd_v, cjj[r]) for r in range(C)]
            return ms, rs

        def merged_step(t, wp, op_cur, op_prev, mean_p, rstd_p,
                        gate_wait=True, gate_prefetch=True):
            if gate_wait:
                @pl.when(t >= 3)
                def _():
                    out_copy(t, op_cur).wait()
            word_gather(t, wp).wait()
            pos_copy(t, wp).wait()

            ms, rs = splats(mean_p, rstd_p)
            accs = []
            for q0 in (0, QR, 2 * QR, 3 * QR):
                vss, vqs = merged_quarter(wp, op_cur, op_prev,
                                          ms[q0:q0 + QR], rs[q0:q0 + QR],
                                          q0)
                accs.extend(zip(vss, vqs))

            out_copy(t - 1, op_prev).start()
            if gate_prefetch:
                @pl.when(t <= n_g - 3)
                def _():
                    start_gather(t + 2, wp)
            return batched_stats(accs)

        start_gather(0, 0)
        start_gather(1, 1)

        word_gather(0, 0).wait()
        pos_copy(0, 0).wait()
        accs0 = []
        for q0 in (0, QR, 2 * QR, 3 * QR):
            vss, vqs = p1_quarter(0, 0, q0)
            accs0.extend(zip(vss, vqs))
        start_gather(2, 0)
        mean, rstd = batched_stats(accs0)

        def outer(go, carry):
            mean_p, rstd_p = carry
            for u in range(6):
                t = 1 + 6 * go + u
                mean_p, rstd_p = merged_step(
                    t, (1 + u) % 2, (1 + u) % 3, u % 3, mean_p, rstd_p)
            return mean_p, rstd_p

        mean, rstd = lax.fori_loop(0, 5, outer, (mean, rstd))

        mean, rstd = merged_step(jnp.int32(n_g - 1), 1, 1, 0, mean, rstd,
                                 gate_wait=True, gate_prefetch=False)

        ms, rs = splats(mean, rstd)
        for q0 in (0, QR, 2 * QR, 3 * QR):
            p2_quarter(1, ms[q0:q0 + QR], rs[q0:q0 + QR], q0)
        out_copy(n_g - 1, 1).start()

        out_copy(n_g - 3, 2).wait()
        out_copy(n_g - 2, 0).wait()
        out_copy(n_g - 1, 1).wait()

    return k


def kernel(input_ids, word_table, pos_table, type_table, ln_weight, ln_bias):
    b, s = input_ids.shape
    ids_flat = jnp.reshape(input_ids.astype(jnp.int32), (b * s,))
    type_row = jnp.reshape(type_table, (D,))
    k = _make_sc_kernel(b * s)
    out = k(ids_flat, word_table, pos_table, type_row, ln_weight, ln_bias)
    return jnp.reshape(out, (b, s, D))


# speedup vs baseline: 12.1040x; 7.5967x over previous
import functools

import jax
import jax.numpy as jnp
from jax import lax
from jax.experimental import pallas as pl
from jax.experimental.pallas import tpu as pltpu
from jax.experimental.pallas import tpu_sc as plsc

D = 1024
EPS = 1e-05
SEQ = 4096
C = 32
TC_BLOCK = 256


@functools.lru_cache(maxsize=None)
def _make_gather_kernel(n_rows):
    info = plsc.get_sparse_core_info()
    nw = info.num_cores * info.num_subcores
    per_w = n_rows // nw
    n_g = per_w // C
    mesh = plsc.VectorSubcoreMesh(core_axis_name="c", subcore_axis_name="s")

    @functools.partial(
        pl.kernel,
        mesh=mesh,
        out_type=jax.ShapeDtypeStruct((n_rows, D), jnp.float32),
        compiler_params=pltpu.CompilerParams(needs_layout_passes=False),
        scratch_types=[
            pltpu.VMEM((per_w,), jnp.int32),
            pltpu.VMEM((C, D), jnp.float32),
            pltpu.VMEM((C, D), jnp.float32),
            pltpu.SemaphoreType.DMA,
            pltpu.SemaphoreType.DMA,
            pltpu.SemaphoreType.DMA,
            pltpu.SemaphoreType.DMA,
        ],
    )
    def k(ids_hbm, word_hbm, out_hbm, idx_v, buf0, buf1,
          semg0, semg1, semo0, semo1):
        bufs = (buf0, buf1)
        semg = (semg0, semg1)
        semo = (semo0, semo1)

        wid = lax.axis_index("s") * info.num_cores + lax.axis_index("c")
        base = wid * per_w
        pltpu.sync_copy(ids_hbm.at[pl.ds(base, per_w)], idx_v)

        def gather(g, p):
            row0 = pl.multiple_of(g * C, C)
            return pltpu.make_async_copy(
                word_hbm.at[idx_v.at[pl.ds(row0, C)]], bufs[p], semg[p])

        def writeout(g, p):
            row0 = pl.multiple_of(base + g * C, C)
            return pltpu.make_async_copy(
                bufs[p], out_hbm.at[pl.ds(row0, C)], semo[p])

        gather(0, 0).start()
        gather(1, 1).start()

        def step(g, p):
            gather(g, p).wait()
            writeout(g, p).start()
            writeout(g, p).wait()

            @pl.when(g + 2 < n_g)
            def _():
                gather(g + 2, p).start()

        def outer(go, carry):
            step(2 * go, 0)
            step(2 * go + 1, 1)
            return carry

        lax.fori_loop(0, n_g // 2, outer, 0)

    return k


def _ln_body(g_ref, p_ref, t_ref, w_ref, b_ref, o_ref):
    x = g_ref[...] + p_ref[...] + t_ref[...]
    mean = jnp.mean(x, axis=-1, keepdims=True)
    cx = x - mean
    var = jnp.mean(cx * cx, axis=-1, keepdims=True)
    y = cx * lax.rsqrt(var + EPS)
    o_ref[...] = y * w_ref[...] + b_ref[...]


@functools.lru_cache(maxsize=None)
def _make_ln_kernel(n_rows):
    n_blocks = n_rows // TC_BLOCK
    pos_blocks = SEQ // TC_BLOCK
    return pl.pallas_call(
        _ln_body,
        grid=(n_blocks,),
        in_specs=[
            pl.BlockSpec((TC_BLOCK, D), lambda i: (i, 0)),
            pl.BlockSpec((TC_BLOCK, D), lambda i: (i % pos_blocks, 0)),
            pl.BlockSpec((1, D), lambda i: (0, 0)),
            pl.BlockSpec((1, D), lambda i: (0, 0)),
            pl.BlockSpec((1, D), lambda i: (0, 0)),
        ],
        out_specs=pl.BlockSpec((TC_BLOCK, D), lambda i: (i, 0)),
        out_shape=jax.ShapeDtypeStruct((n_rows, D), jnp.float32),
    )


def kernel(input_ids, word_table, pos_table, type_table, ln_weight, ln_bias):
    b, s = input_ids.shape
    n = b * s
    ids_flat = jnp.reshape(input_ids.astype(jnp.int32), (n,))
    gathered = _make_gather_kernel(n)(ids_flat, word_table)
    out = _make_ln_kernel(n)(
        gathered, pos_table[:SEQ], jnp.reshape(type_table, (1, D)),
        jnp.reshape(ln_weight, (1, D)), jnp.reshape(ln_bias, (1, D)))
    return jnp.reshape(out, (b, s, D))
